# Initial kernel scaffold; baseline (speedup 1.0000x reference)
import jax
import jax.numpy as jnp
from jax.experimental import pallas as pl


def kernel(x, We, be, bn_g, bn_b, W_gat, att_src, att_dst, b_gat, ln1_g, ln1_b, W1, b1, W2, b2, ln2_g, ln2_b, Wc, bc, edge_index):
    raise NotImplementedError("write your pallas kernel here")


# speedup vs baseline: 10.2583x; 10.2583x over previous
import functools

import jax
import jax.numpy as jnp
from jax import lax
from jax.experimental import pallas as pl
from jax.experimental.pallas import tpu as pltpu
from jax.experimental.pallas import tpu_sc as plsc

N = 10000
E = 320000
NMOD = 3
IN_DIM = 200
D = 128
L = 3
ETOT = E + N
EP = 360448
NIROWS = EP // 128
CH = EP // 32
CHROWS = CH // 128
NHALF = N // 2
ACCR = 5120
ASTRIPE = ACCR // 16
NPAD = 10240
STRIPE = NPAD // 16
RB = 1000
NBLK = N // RB

_f32 = jnp.float32
_i32 = jnp.int32


def _embed_body(x_ref, w_ref, be_ref, h_ref, s_ref, ss_ref):
    h = jnp.dot(x_ref[...], w_ref[...], preferred_element_type=_f32)
    h = h + (be_ref[0, :] + be_ref[1, :] + be_ref[2, :])[None, :]
    h_ref[...] = h

    @pl.when(pl.program_id(0) == 0)
    def _():
        s_ref[...] = jnp.zeros_like(s_ref)
        ss_ref[...] = jnp.zeros_like(ss_ref)

    s_ref[...] += jnp.sum(h, axis=0, keepdims=True)
    ss_ref[...] += jnp.sum(h * h, axis=0, keepdims=True)


def _embed(x2, w_all, be):
    return pl.pallas_call(
        _embed_body,
        grid=(NBLK,),
        in_specs=[
            pl.BlockSpec((RB, NMOD * IN_DIM), lambda i: (i, 0)),
            pl.BlockSpec((NMOD * IN_DIM, D), lambda i: (0, 0)),
            pl.BlockSpec((NMOD, D), lambda i: (0, 0)),
        ],
        out_specs=[
            pl.BlockSpec((RB, D), lambda i: (i, 0)),
            pl.BlockSpec((1, D), lambda i: (0, 0)),
            pl.BlockSpec((1, D), lambda i: (0, 0)),
        ],
        out_shape=[
            jax.ShapeDtypeStruct((N, D), _f32),
            jax.ShapeDtypeStruct((1, D), _f32),
            jax.ShapeDtypeStruct((1, D), _f32),
        ],
    )(x2, w_all, be)


def _bn_gat_body(hp_ref, s_ref, ss_ref, g_ref, b_ref, w_ref, as_ref, ad_ref,
                 h_ref, xw_ref, aso_ref, ado_ref):
    mu = s_ref[...] / N
    var = ss_ref[...] / N - mu * mu
    h = (hp_ref[...] - mu) * jax.lax.rsqrt(var + 1e-5) * g_ref[...] + b_ref[...]
    h = jnp.maximum(h, 0.0)
    h_ref[...] = h
    xw = jnp.dot(h, w_ref[...], preferred_element_type=_f32)
    xw_ref[...] = xw
    aso_ref[...] = jnp.sum(xw * as_ref[...], axis=1, keepdims=True)
    ado_ref[...] = jnp.sum(xw * ad_ref[...], axis=1, keepdims=True)


def _bn_gat(h_pre, s, ss, bn_g, bn_b, wgat, att_s, att_d):
    row = lambda i: (i, 0)
    full = lambda i: (0, 0)
    return pl.pallas_call(
        _bn_gat_body,
        grid=(NBLK,),
        in_specs=[
            pl.BlockSpec((RB, D), row),
            pl.BlockSpec((1, D), full), pl.BlockSpec((1, D), full),
            pl.BlockSpec((1, D), full), pl.BlockSpec((1, D), full),
            pl.BlockSpec((D, D), full),
            pl.BlockSpec((1, D), full), pl.BlockSpec((1, D), full),
        ],
        out_specs=[
            pl.BlockSpec((RB, D), row), pl.BlockSpec((RB, D), row),
            pl.BlockSpec((RB, 1), row), pl.BlockSpec((RB, 1), row),
        ],
        out_shape=[
            jax.ShapeDtypeStruct((N, D), _f32),
            jax.ShapeDtypeStruct((N, D), _f32),
            jax.ShapeDtypeStruct((N, 1), _f32),
            jax.ShapeDtypeStruct((N, 1), _f32),
        ],
    )(h_pre, s, ss, bn_g, bn_b, wgat, att_s, att_d)


def _ln(t, g, b):
    mu = jnp.mean(t, axis=1, keepdims=True)
    v = jnp.mean((t - mu) ** 2, axis=1, keepdims=True)
    return (t - mu) * jax.lax.rsqrt(v + 1e-5) * g + b


def _ffn_block(h, xp0_ref, xp1_ref, d0_ref, d1_ref, bg_ref, l1g_ref, l1b_ref,
               w1_ref, b1_ref, w2_ref, b2_ref, l2g_ref, l2b_ref):
    ivd = 1.0 / (d0_ref[:, 0] + d1_ref[:, 0] + 1e-16)
    xatt = (xp0_ref[...] + xp1_ref[...]) * ivd[:, None] + bg_ref[...]
    h1 = _ln(h + xatt, l1g_ref[...], l1b_ref[...])
    f = jnp.dot(jnp.maximum(
        jnp.dot(h1, w1_ref[...], preferred_element_type=_f32) + b1_ref[...],
        0.0), w2_ref[...], preferred_element_type=_f32) + b2_ref[...]
    return _ln(h1 + f, l2g_ref[...], l2b_ref[...]), ivd


def _layer_mid_body(h_ref, xp0_ref, xp1_ref, d0_ref, d1_ref, bg_ref,
                    l1g_ref, l1b_ref, w1_ref, b1_ref, w2_ref, b2_ref,
                    l2g_ref, l2b_ref, w_ref, as_ref, ad_ref,
                    hn_ref, xw_ref, aso_ref, ado_ref, ivd_ref):
    hn, ivd = _ffn_block(h_ref[...], xp0_ref, xp1_ref, d0_ref, d1_ref, bg_ref,
                         l1g_ref, l1b_ref, w1_ref, b1_ref, w2_ref, b2_ref,
                         l2g_ref, l2b_ref)
    hn_ref[...] = hn
    ivd_ref[...] = ivd[:, None]
    xw = jnp.dot(hn, w_ref[...], preferred_element_type=_f32)
    xw_ref[...] = xw
    aso_ref[...] = jnp.sum(xw * as_ref[...], axis=1, keepdims=True)
    ado_ref[...] = jnp.sum(xw * ad_ref[...], axis=1, keepdims=True)


def _layer_mid(h, xp0, xp1, d0, d1, bg, l1g, l1b, w1, b1, w2, b2, l2g, l2b,
               wgat, att_s, att_d):
    row = lambda i: (i, 0)
    full = lambda i: (0, 0)
    return pl.pallas_call(
        _layer_mid_body,
        grid=(NBLK,),
        in_specs=[
            pl.BlockSpec((RB, D), row), pl.BlockSpec((RB, D), row),
            pl.BlockSpec((RB, D), row),
            pl.BlockSpec((RB, 1), row), pl.BlockSpec((RB, 1), row),
            pl.BlockSpec((1, D), full), pl.BlockSpec((1, D), full),
            pl.BlockSpec((1, D), full),
            pl.BlockSpec((D, D), full), pl.BlockSpec((1, D), full),
            pl.BlockSpec((D, D), full), pl.BlockSpec((1, D), full),
            pl.BlockSpec((1, D), full), pl.BlockSpec((1, D), full),
            pl.BlockSpec((D, D), full), pl.BlockSpec((1, D), full),
            pl.BlockSpec((1, D), full),
        ],
        out_specs=[
            pl.BlockSpec((RB, D), row), pl.BlockSpec((RB, D), row),
            pl.BlockSpec((RB, 1), row), pl.BlockSpec((RB, 1), row),
            pl.BlockSpec((RB, 1), row),
        ],
        out_shape=[
            jax.ShapeDtypeStruct((N, D), _f32),
            jax.ShapeDtypeStruct((N, D), _f32),
            jax.ShapeDtypeStruct((N, 1), _f32),
            jax.ShapeDtypeStruct((N, 1), _f32),
            jax.ShapeDtypeStruct((N, 1), _f32),
        ],
    )(h, xp0, xp1, d0, d1, bg, l1g, l1b, w1, b1, w2, b2, l2g, l2b,
      wgat, att_s, att_d)


def _layer_last_body(h_ref, xp0_ref, xp1_ref, d0_ref, d1_ref, bg_ref,
                     l1g_ref, l1b_ref, w1_ref, b1_ref, w2_ref, b2_ref,
                     l2g_ref, l2b_ref, wc_ref, bc_ref, out_ref, ivd_ref):
    hn, ivd = _ffn_block(h_ref[...], xp0_ref, xp1_ref, d0_ref, d1_ref, bg_ref,
                         l1g_ref, l1b_ref, w1_ref, b1_ref, w2_ref, b2_ref,
                         l2g_ref, l2b_ref)
    ivd_ref[...] = ivd[:, None]
    out_ref[...] = jnp.dot(hn, wc_ref[...], preferred_element_type=_f32) + bc_ref[...]


def _layer_last(h, xp0, xp1, d0, d1, bg, l1g, l1b, w1, b1, w2, b2, l2g, l2b,
                wc, bc):
    row = lambda i: (i, 0)
    full = lambda i: (0, 0)
    return pl.pallas_call(
        _layer_last_body,
        grid=(NBLK,),
        in_specs=[
            pl.BlockSpec((RB, D), row), pl.BlockSpec((RB, D), row),
            pl.BlockSpec((RB, D), row),
            pl.BlockSpec((RB, 1), row), pl.BlockSpec((RB, 1), row),
            pl.BlockSpec((1, D), full), pl.BlockSpec((1, D), full),
            pl.BlockSpec((1, D), full),
            pl.BlockSpec((D, D), full), pl.BlockSpec((1, D), full),
            pl.BlockSpec((D, D), full), pl.BlockSpec((1, D), full),
            pl.BlockSpec((1, D), full), pl.BlockSpec((1, D), full),
            pl.BlockSpec((D, 1), full), pl.BlockSpec((1, 1), full),
        ],
        out_specs=[
            pl.BlockSpec((RB, 1), row),
            pl.BlockSpec((RB, 1), row),
        ],
        out_shape=[
            jax.ShapeDtypeStruct((N, 1), _f32),
            jax.ShapeDtypeStruct((N, 1), _f32),
        ],
    )(h, xp0, xp1, d0, d1, bg, l1g, l1b, w1, b1, w2, b2, l2g, l2b, wc, bc)


def _scale_rows(rowsb, eeb, j):
    @pl.loop(0, 8)
    def _(g):
        a16 = eeb[j, pl.ds(g * 16, 16)]
        for kk in range(16):
            av = a16[jnp.full((16,), kk, _i32)]
            r = g * 16 + kk
            for c in range(8):
                sl = pl.ds(c * 16, 16)
                rowsb[0, r, sl] = rowsb[0, r, sl] * av


def _zero_stripes(rowsb, xatt_sh, sid, den_sh):
    z16 = jnp.zeros((16,), _f32)

    @pl.loop(0, 128)
    def _(i):
        for c in range(8):
            rowsb[0, i, pl.ds(c * 16, 16)] = z16

    abase = sid * ASTRIPE
    pltpu.sync_copy(rowsb.at[0], xatt_sh.at[pl.ds(abase, 128)])
    pltpu.sync_copy(rowsb.at[0], xatt_sh.at[pl.ds(abase + 128, 128)])
    pltpu.sync_copy(rowsb.at[0, pl.ds(0, 64)],
                    xatt_sh.at[pl.ds(abase + 256, 64)])
    if den_sh is not None:
        for k in range(STRIPE // 128):
            pltpu.sync_copy(rowsb.at[0, k],
                            den_sh.at[pl.ds(sid * STRIPE + k * 128, 128)])


def _sc_edge_a_body(as_hbm, ad_hbm, src2_hbm, dst2_hbm, xw_hbm,
                    ee_out, den0_out, den1_out, xatt0_out, xatt1_out,
                    idx_s, idx_d, idx_t, ase, eeb, rowsb,
                    den_sh, xatt_sh, sem, sem2):
    cid = lax.axis_index("c")
    sid = lax.axis_index("s")
    wid = sid * 2 + cid
    iota = lax.iota(_i32, 16)
    z16 = jnp.zeros((16,), _f32)

    _zero_stripes(rowsb, xatt_sh, sid, den_sh)

    rbase = wid * CHROWS
    pltpu.sync_copy(src2_hbm.at[pl.ds(rbase, CHROWS)], idx_s)
    pltpu.sync_copy(dst2_hbm.at[pl.ds(rbase, CHROWS)], idx_d)

    @pl.loop(0, CHROWS)
    def _(j):
        c1 = pltpu.async_copy(as_hbm.at[idx_s.at[j]], ase.at[j], sem)
        c2 = pltpu.async_copy(ad_hbm.at[idx_d.at[j]], eeb.at[j], sem2)
        c1.wait()
        c2.wait()

    @pl.loop(0, CHROWS)
    def _(r):
        gbase = rbase * 128 + r * 128
        for c in range(8):
            sl = pl.ds(c * 16, 16)
            ev = ase[r, sl] + eeb[r, sl]
            ev = jnp.where(ev > 0, ev, 0.2 * ev)
            ee = jnp.exp(ev)
            idv = jnp.full((16,), gbase + c * 16, _i32) + iota
            eeb[r, sl] = jnp.where(idv < ETOT, ee, z16)
            d16 = idx_d[r, sl]
            idx_t[r, sl] = jnp.where(d16 < NHALF, d16,
                                     jnp.full((16,), NHALF, _i32))

    pltpu.sync_copy(eeb, ee_out.at[pl.ds(rbase, CHROWS)])

    plsc.subcore_barrier()

    @pl.loop(0, CHROWS)
    def _(j):
        pltpu.sync_copy(eeb.at[j], den_sh.at[idx_d.at[j]], add=True)

    @pl.loop(0, CHROWS)
    def _(j):
      @pl.when(rbase * 128 + j * 128 < ETOT)
      def _():
        pltpu.async_copy(xw_hbm.at[idx_s.at[j]], rowsb.at[0], sem).wait()
        _scale_rows(rowsb, eeb, j)
        pltpu.sync_copy(rowsb.at[0], xatt_sh.at[idx_t.at[j]], add=True)

    plsc.subcore_barrier()

    abase = sid * ASTRIPE

    @pl.when(cid == 0)
    def _():
        pltpu.sync_copy(den_sh.at[pl.ds(sid * STRIPE, STRIPE)],
                        den0_out.at[pl.ds(sid * STRIPE, STRIPE)])
        pltpu.sync_copy(xatt_sh.at[pl.ds(abase, ASTRIPE)],
                        xatt0_out.at[pl.ds(abase, ASTRIPE)])

    @pl.when(cid == 1)
    def _():
        pltpu.sync_copy(den_sh.at[pl.ds(sid * STRIPE, STRIPE)],
                        den1_out.at[pl.ds(sid * STRIPE, STRIPE)])
        pltpu.sync_copy(xatt_sh.at[pl.ds(abase, ASTRIPE)],
                        xatt1_out.at[pl.ds(abase, ASTRIPE)])


def _sc_edge_b_body(src2_hbm, dst2_hbm, xw_hbm, ee_hbm,
                    xatt0_out, xatt1_out,
                    idx_s, idx_t, eeb, rowsb, xatt_sh, sem):
    cid = lax.axis_index("c")
    sid = lax.axis_index("s")
    wid = sid * 2 + cid

    _zero_stripes(rowsb, xatt_sh, sid, None)

    rbase = wid * CHROWS
    pltpu.sync_copy(src2_hbm.at[pl.ds(rbase, CHROWS)], idx_s)
    pltpu.sync_copy(ee_hbm.at[pl.ds(rbase, CHROWS)], eeb)
    pltpu.sync_copy(dst2_hbm.at[pl.ds(rbase, CHROWS)], idx_t)

    @pl.loop(0, CHROWS)
    def _(r):
        for c in range(8):
            sl = pl.ds(c * 16, 16)
            d16 = idx_t[r, sl] - NHALF
            ok = (d16 >= 0) & (d16 < NHALF)
            idx_t[r, sl] = jnp.where(ok, d16, jnp.full((16,), NHALF, _i32))

    plsc.subcore_barrier()

    @pl.loop(0, CHROWS)
    def _(j):
      @pl.when(rbase * 128 + j * 128 < ETOT)
      def _():
        pltpu.async_copy(xw_hbm.at[idx_s.at[j]], rowsb.at[0], sem).wait()
        _scale_rows(rowsb, eeb, j)
        pltpu.sync_copy(rowsb.at[0], xatt_sh.at[idx_t.at[j]], add=True)

    plsc.subcore_barrier()

    abase = sid * ASTRIPE

    @pl.when(cid == 0)
    def _():
        pltpu.sync_copy(xatt_sh.at[pl.ds(abase, ASTRIPE)],
                        xatt0_out.at[pl.ds(abase, ASTRIPE)])

    @pl.when(cid == 1)
    def _():
        pltpu.sync_copy(xatt_sh.at[pl.ds(abase, ASTRIPE)],
                        xatt1_out.at[pl.ds(abase, ASTRIPE)])


@functools.cache
def _sc_edge_a():
    mesh = plsc.VectorSubcoreMesh(core_axis_name="c", subcore_axis_name="s")
    return pl.kernel(
        _sc_edge_a_body,
        out_type=[
            jax.ShapeDtypeStruct((NIROWS, 128), _f32),
            jax.ShapeDtypeStruct((NPAD,), _f32),
            jax.ShapeDtypeStruct((NPAD,), _f32),
            jax.ShapeDtypeStruct((ACCR, D), _f32),
            jax.ShapeDtypeStruct((ACCR, D), _f32),
        ],
        mesh=mesh,
        scratch_types=[
            pltpu.VMEM((CHROWS, 128), _i32),
            pltpu.VMEM((CHROWS, 128), _i32),
            pltpu.VMEM((CHROWS, 128), _i32),
            pltpu.VMEM((CHROWS, 128), _f32),
            pltpu.VMEM((CHROWS, 128), _f32),
            pltpu.VMEM((2, 128, D), _f32),
            pltpu.VMEM_SHARED((NPAD,), _f32),
            pltpu.VMEM_SHARED((ACCR, D), _f32),
            pltpu.SemaphoreType.DMA,
            pltpu.SemaphoreType.DMA,
        ],
    )


@functools.cache
def _sc_edge_b():
    mesh = plsc.VectorSubcoreMesh(core_axis_name="c", subcore_axis_name="s")
    return pl.kernel(
        _sc_edge_b_body,
        out_type=[
            jax.ShapeDtypeStruct((ACCR, D), _f32),
            jax.ShapeDtypeStruct((ACCR, D), _f32),
        ],
        mesh=mesh,
        scratch_types=[
            pltpu.VMEM((CHROWS, 128), _i32),
            pltpu.VMEM((CHROWS, 128), _i32),
            pltpu.VMEM((CHROWS, 128), _f32),
            pltpu.VMEM((2, 128, D), _f32),
            pltpu.VMEM_SHARED((ACCR, D), _f32),
            pltpu.SemaphoreType.DMA,
        ],
    )


def _sc_alpha_body(ee_hbm, ivd_hbm, dst2_hbm, alpha_out, idx_d, eev, ivv, sem):
    cid = lax.axis_index("c")
    sid = lax.axis_index("s")
    wid = sid * 2 + cid
    rbase = wid * CHROWS
    pltpu.sync_copy(dst2_hbm.at[pl.ds(rbase, CHROWS)], idx_d)
    pltpu.sync_copy(ee_hbm.at[pl.ds(rbase, CHROWS)], eev)

    @pl.loop(0, CHROWS)
    def _(j):
        pltpu.async_copy(ivd_hbm.at[idx_d.at[j]], ivv.at[j], sem).wait()

    @pl.loop(0, CHROWS)
    def _(r):
        for c in range(8):
            sl = pl.ds(c * 16, 16)
            eev[r, sl] = eev[r, sl] * ivv[r, sl]

    pltpu.sync_copy(eev, alpha_out.at[pl.ds(rbase, CHROWS)])


@functools.cache
def _sc_alpha():
    mesh = plsc.VectorSubcoreMesh(core_axis_name="c", subcore_axis_name="s")
    return pl.kernel(
        _sc_alpha_body,
        out_type=jax.ShapeDtypeStruct((NIROWS, 128), _f32),
        mesh=mesh,
        scratch_types=[
            pltpu.VMEM((CHROWS, 128), _i32),
            pltpu.VMEM((CHROWS, 128), _f32),
            pltpu.VMEM((CHROWS, 128), _f32),
            pltpu.SemaphoreType.DMA,
        ],
    )


def kernel(x, We, be, bn_g, bn_b, W_gat, att_src, att_dst, b_gat,
           ln1_g, ln1_b, W1, b1, W2, b2, ln2_g, ln2_b, Wc, bc, edge_index):
    x2 = x.reshape(N, NMOD * IN_DIM)
    w_all = We.reshape(NMOD * IN_DIM, D)

    loop = jnp.arange(N, dtype=edge_index.dtype)
    pad = jnp.zeros((EP - ETOT,), dtype=edge_index.dtype)
    src2 = jnp.concatenate([edge_index[0], loop, pad]).reshape(NIROWS, 128)
    dst2 = jnp.concatenate([edge_index[1], loop, pad]).reshape(NIROWS, 128)

    h_pre, s, ss = _embed(x2, w_all, be)
    h, xw, a_s, a_d = _bn_gat(h_pre, s, ss, bn_g[None, :], bn_b[None, :],
                              W_gat[0], att_src[0].reshape(1, D),
                              att_dst[0].reshape(1, D))

    sc_a = _sc_edge_a()
    sc_b = _sc_edge_b()
    sc_alpha = _sc_alpha()
    alphas = []
    for l in range(L):
        ee, den0, den1, a0, a1 = sc_a(a_s.reshape(N), a_d.reshape(N), src2,
                                      dst2, xw)
        q0, q1 = sc_b(src2, dst2, xw, ee)
        xp0 = jnp.concatenate([a0[:NHALF], q0[:NHALF]], axis=0)
        xp1 = jnp.concatenate([a1[:NHALF], q1[:NHALF]], axis=0)
        args = (h, xp0, xp1, den0[:N, None], den1[:N, None],
                b_gat[l][None, :], ln1_g[l][None, :], ln1_b[l][None, :],
                W1[l], b1[l][None, :], W2[l], b2[l][None, :],
                ln2_g[l][None, :], ln2_b[l][None, :])
        if l < L - 1:
            h, xw, a_s, a_d, ivd = _layer_mid(
                *args, W_gat[l + 1], att_src[l + 1].reshape(1, D),
                att_dst[l + 1].reshape(1, D))
        else:
            logits, ivd = _layer_last(*args, Wc, bc[None, :])
        alpha = sc_alpha(ee, ivd.reshape(N), dst2)
        alphas.append(alpha.reshape(EP)[:ETOT].reshape(ETOT, 1))

    return (logits.reshape(N), alphas[0], alphas[1], alphas[2])


# speedup vs baseline: 10.9094x; 1.0635x over previous
import functools

import jax
import jax.numpy as jnp
from jax import lax
from jax.experimental import pallas as pl
from jax.experimental.pallas import tpu as pltpu
from jax.experimental.pallas import tpu_sc as plsc

N = 10000
E = 320000
NMOD = 3
IN_DIM = 200
D = 128
L = 3
ETOT = E + N
EP = 360448
NIROWS = EP // 128
CH = EP // 32
CHROWS = CH // 128
NHALF = N // 2
ACCR = 5120
ASTRIPE = ACCR // 16
NPAD = 10240
STRIPE = NPAD // 16
RB = 1000
NBLK = N // RB

_f32 = jnp.float32
_i32 = jnp.int32


def _embed_body(x_ref, w_ref, be_ref, h_ref, s_ref, ss_ref):
    h = jnp.dot(x_ref[...], w_ref[...], preferred_element_type=_f32)
    h = h + (be_ref[0, :] + be_ref[1, :] + be_ref[2, :])[None, :]
    h_ref[...] = h

    @pl.when(pl.program_id(0) == 0)
    def _():
        s_ref[...] = jnp.zeros_like(s_ref)
        ss_ref[...] = jnp.zeros_like(ss_ref)

    s_ref[...] += jnp.sum(h, axis=0, keepdims=True)
    ss_ref[...] += jnp.sum(h * h, axis=0, keepdims=True)


def _embed(x2, w_all, be):
    return pl.pallas_call(
        _embed_body,
        grid=(NBLK,),
        in_specs=[
            pl.BlockSpec((RB, NMOD * IN_DIM), lambda i: (i, 0)),
            pl.BlockSpec((NMOD * IN_DIM, D), lambda i: (0, 0)),
            pl.BlockSpec((NMOD, D), lambda i: (0, 0)),
        ],
        out_specs=[
            pl.BlockSpec((RB, D), lambda i: (i, 0)),
            pl.BlockSpec((1, D), lambda i: (0, 0)),
            pl.BlockSpec((1, D), lambda i: (0, 0)),
        ],
        out_shape=[
            jax.ShapeDtypeStruct((N, D), _f32),
            jax.ShapeDtypeStruct((1, D), _f32),
            jax.ShapeDtypeStruct((1, D), _f32),
        ],
    )(x2, w_all, be)


def _bn_gat_body(hp_ref, s_ref, ss_ref, g_ref, b_ref, w_ref, as_ref, ad_ref,
                 h_ref, xw_ref, aso_ref, ado_ref):
    mu = s_ref[...] / N
    var = ss_ref[...] / N - mu * mu
    h = (hp_ref[...] - mu) * jax.lax.rsqrt(var + 1e-5) * g_ref[...] + b_ref[...]
    h = jnp.maximum(h, 0.0)
    h_ref[...] = h
    xw = jnp.dot(h, w_ref[...], preferred_element_type=_f32)
    xw_ref[...] = xw
    aso_ref[...] = jnp.sum(xw * as_ref[...], axis=1, keepdims=True)
    ado_ref[...] = jnp.sum(xw * ad_ref[...], axis=1, keepdims=True)


def _bn_gat(h_pre, s, ss, bn_g, bn_b, wgat, att_s, att_d):
    row = lambda i: (i, 0)
    full = lambda i: (0, 0)
    return pl.pallas_call(
        _bn_gat_body,
        grid=(NBLK,),
        in_specs=[
            pl.BlockSpec((RB, D), row),
            pl.BlockSpec((1, D), full), pl.BlockSpec((1, D), full),
            pl.BlockSpec((1, D), full), pl.BlockSpec((1, D), full),
            pl.BlockSpec((D, D), full),
            pl.BlockSpec((1, D), full), pl.BlockSpec((1, D), full),
        ],
        out_specs=[
            pl.BlockSpec((RB, D), row), pl.BlockSpec((RB, D), row),
            pl.BlockSpec((RB, 1), row), pl.BlockSpec((RB, 1), row),
        ],
        out_shape=[
            jax.ShapeDtypeStruct((N, D), _f32),
            jax.ShapeDtypeStruct((N, D), _f32),
            jax.ShapeDtypeStruct((N, 1), _f32),
            jax.ShapeDtypeStruct((N, 1), _f32),
        ],
    )(h_pre, s, ss, bn_g, bn_b, wgat, att_s, att_d)


def _ln(t, g, b):
    mu = jnp.mean(t, axis=1, keepdims=True)
    v = jnp.mean((t - mu) ** 2, axis=1, keepdims=True)
    return (t - mu) * jax.lax.rsqrt(v + 1e-5) * g + b


def _ffn_block(h, xp0_ref, xp1_ref, d0_ref, d1_ref, bg_ref, l1g_ref, l1b_ref,
               w1_ref, b1_ref, w2_ref, b2_ref, l2g_ref, l2b_ref):
    ivd = 1.0 / (d0_ref[:, 0] + d1_ref[:, 0] + 1e-16)
    xatt = (xp0_ref[...] + xp1_ref[...]) * ivd[:, None] + bg_ref[...]
    h1 = _ln(h + xatt, l1g_ref[...], l1b_ref[...])
    f = jnp.dot(jnp.maximum(
        jnp.dot(h1, w1_ref[...], preferred_element_type=_f32) + b1_ref[...],
        0.0), w2_ref[...], preferred_element_type=_f32) + b2_ref[...]
    return _ln(h1 + f, l2g_ref[...], l2b_ref[...]), ivd


def _layer_mid_body(h_ref, xp0_ref, xp1_ref, d0_ref, d1_ref, bg_ref,
                    l1g_ref, l1b_ref, w1_ref, b1_ref, w2_ref, b2_ref,
                    l2g_ref, l2b_ref, w_ref, as_ref, ad_ref,
                    hn_ref, xw_ref, aso_ref, ado_ref, ivd_ref):
    hn, ivd = _ffn_block(h_ref[...], xp0_ref, xp1_ref, d0_ref, d1_ref, bg_ref,
                         l1g_ref, l1b_ref, w1_ref, b1_ref, w2_ref, b2_ref,
                         l2g_ref, l2b_ref)
    hn_ref[...] = hn
    ivd_ref[...] = ivd[:, None]
    xw = jnp.dot(hn, w_ref[...], preferred_element_type=_f32)
    xw_ref[...] = xw
    aso_ref[...] = jnp.sum(xw * as_ref[...], axis=1, keepdims=True)
    ado_ref[...] = jnp.sum(xw * ad_ref[...], axis=1, keepdims=True)


def _layer_mid(h, xp0, xp1, d0, d1, bg, l1g, l1b, w1, b1, w2, b2, l2g, l2b,
               wgat, att_s, att_d):
    row = lambda i: (i, 0)
    full = lambda i: (0, 0)
    return pl.pallas_call(
        _layer_mid_body,
        grid=(NBLK,),
        in_specs=[
            pl.BlockSpec((RB, D), row), pl.BlockSpec((RB, D), row),
            pl.BlockSpec((RB, D), row),
            pl.BlockSpec((RB, 1), row), pl.BlockSpec((RB, 1), row),
            pl.BlockSpec((1, D), full), pl.BlockSpec((1, D), full),
            pl.BlockSpec((1, D), full),
            pl.BlockSpec((D, D), full), pl.BlockSpec((1, D), full),
            pl.BlockSpec((D, D), full), pl.BlockSpec((1, D), full),
            pl.BlockSpec((1, D), full), pl.BlockSpec((1, D), full),
            pl.BlockSpec((D, D), full), pl.BlockSpec((1, D), full),
            pl.BlockSpec((1, D), full),
        ],
        out_specs=[
            pl.BlockSpec((RB, D), row), pl.BlockSpec((RB, D), row),
            pl.BlockSpec((RB, 1), row), pl.BlockSpec((RB, 1), row),
            pl.BlockSpec((RB, 1), row),
        ],
        out_shape=[
            jax.ShapeDtypeStruct((N, D), _f32),
            jax.ShapeDtypeStruct((N, D), _f32),
            jax.ShapeDtypeStruct((N, 1), _f32),
            jax.ShapeDtypeStruct((N, 1), _f32),
            jax.ShapeDtypeStruct((N, 1), _f32),
        ],
    )(h, xp0, xp1, d0, d1, bg, l1g, l1b, w1, b1, w2, b2, l2g, l2b,
      wgat, att_s, att_d)


def _layer_last_body(h_ref, xp0_ref, xp1_ref, d0_ref, d1_ref, bg_ref,
                     l1g_ref, l1b_ref, w1_ref, b1_ref, w2_ref, b2_ref,
                     l2g_ref, l2b_ref, wc_ref, bc_ref, out_ref, ivd_ref):
    hn, ivd = _ffn_block(h_ref[...], xp0_ref, xp1_ref, d0_ref, d1_ref, bg_ref,
                         l1g_ref, l1b_ref, w1_ref, b1_ref, w2_ref, b2_ref,
                         l2g_ref, l2b_ref)
    ivd_ref[...] = ivd[:, None]
    out_ref[...] = jnp.dot(hn, wc_ref[...], preferred_element_type=_f32) + bc_ref[...]


def _layer_last(h, xp0, xp1, d0, d1, bg, l1g, l1b, w1, b1, w2, b2, l2g, l2b,
                wc, bc):
    row = lambda i: (i, 0)
    full = lambda i: (0, 0)
    return pl.pallas_call(
        _layer_last_body,
        grid=(NBLK,),
        in_specs=[
            pl.BlockSpec((RB, D), row), pl.BlockSpec((RB, D), row),
            pl.BlockSpec((RB, D), row),
            pl.BlockSpec((RB, 1), row), pl.BlockSpec((RB, 1), row),
            pl.BlockSpec((1, D), full), pl.BlockSpec((1, D), full),
            pl.BlockSpec((1, D), full),
            pl.BlockSpec((D, D), full), pl.BlockSpec((1, D), full),
            pl.BlockSpec((D, D), full), pl.BlockSpec((1, D), full),
            pl.BlockSpec((1, D), full), pl.BlockSpec((1, D), full),
            pl.BlockSpec((D, 1), full), pl.BlockSpec((1, 1), full),
        ],
        out_specs=[
            pl.BlockSpec((RB, 1), row),
            pl.BlockSpec((RB, 1), row),
        ],
        out_shape=[
            jax.ShapeDtypeStruct((N, 1), _f32),
            jax.ShapeDtypeStruct((N, 1), _f32),
        ],
    )(h, xp0, xp1, d0, d1, bg, l1g, l1b, w1, b1, w2, b2, l2g, l2b, wc, bc)


def _scale_rows(rowsb, b, eeb, j):
    @pl.loop(0, 8)
    def _(g):
        a16 = eeb[j, pl.ds(g * 16, 16)]
        for kk in range(16):
            av = a16[jnp.full((16,), kk, _i32)]
            r = g * 16 + kk
            for c in range(8):
                sl = pl.ds(c * 16, 16)
                rowsb[b, r, sl] = rowsb[b, r, sl] * av


def _msg_pass(xw_hbm, idx_s, idx_t, eeb, rowsb, xatt_sh, sem, sem2, rbase):
    @pl.loop(0, CHROWS, step=2)
    def _(j0):
        live0 = rbase * 128 + j0 * 128 < ETOT
        live1 = rbase * 128 + (j0 + 1) * 128 < ETOT

        @pl.when(live0)
        def _():
            c0 = pltpu.async_copy(xw_hbm.at[idx_s.at[j0]], rowsb.at[0], sem)

            @pl.when(live1)
            def _():
                pltpu.async_copy(xw_hbm.at[idx_s.at[j0 + 1]], rowsb.at[1],
                                 sem2)

            c0.wait()
            _scale_rows(rowsb, 0, eeb, j0)
            pltpu.sync_copy(rowsb.at[0], xatt_sh.at[idx_t.at[j0]], add=True)

            @pl.when(live1)
            def _():
                pltpu.make_async_copy(xw_hbm.at[idx_s.at[j0 + 1]],
                                      rowsb.at[1], sem2).wait()
                _scale_rows(rowsb, 1, eeb, j0 + 1)
                pltpu.sync_copy(rowsb.at[1], xatt_sh.at[idx_t.at[j0 + 1]],
                                add=True)


def _zero_stripes(rowsb, xatt_sh, sid, den_sh):
    z16 = jnp.zeros((16,), _f32)

    @pl.loop(0, 128)
    def _(i):
        for c in range(8):
            rowsb[0, i, pl.ds(c * 16, 16)] = z16

    abase = sid * ASTRIPE
    pltpu.sync_copy(rowsb.at[0], xatt_sh.at[pl.ds(abase, 128)])
    pltpu.sync_copy(rowsb.at[0], xatt_sh.at[pl.ds(abase + 128, 128)])
    pltpu.sync_copy(rowsb.at[0, pl.ds(0, 64)],
                    xatt_sh.at[pl.ds(abase + 256, 64)])
    if den_sh is not None:
        for k in range(STRIPE // 128):
            pltpu.sync_copy(rowsb.at[0, k],
                            den_sh.at[pl.ds(sid * STRIPE + k * 128, 128)])


def _sc_edge_a_body(as_hbm, ad_hbm, src2_hbm, dst2_hbm, xw_hbm,
                    ee_out, den0_out, den1_out, xatt0_out, xatt1_out,
                    idx_s, idx_d, idx_t, ase, eeb, rowsb,
                    den_sh, xatt_sh, sem, sem2):
    cid = lax.axis_index("c")
    sid = lax.axis_index("s")
    wid = sid * 2 + cid
    iota = lax.iota(_i32, 16)
    z16 = jnp.zeros((16,), _f32)

    _zero_stripes(rowsb, xatt_sh, sid, den_sh)

    rbase = wid * CHROWS
    pltpu.sync_copy(src2_hbm.at[pl.ds(rbase, CHROWS)], idx_s)
    pltpu.sync_copy(dst2_hbm.at[pl.ds(rbase, CHROWS)], idx_d)

    @pl.loop(0, CHROWS, step=8)
    def _(j0):
        cs = []
        for b in range(8):
            cs.append(pltpu.async_copy(as_hbm.at[idx_s.at[j0 + b]],
                                       ase.at[j0 + b], sem))
            cs.append(pltpu.async_copy(ad_hbm.at[idx_d.at[j0 + b]],
                                       eeb.at[j0 + b], sem2))
        for c in cs:
            c.wait()

    @pl.loop(0, CHROWS)
    def _(r):
        gbase = rbase * 128 + r * 128
        for c in range(8):
            sl = pl.ds(c * 16, 16)
            ev = ase[r, sl] + eeb[r, sl]
            ev = jnp.where(ev > 0, ev, 0.2 * ev)
            ee = jnp.exp(ev)
            idv = jnp.full((16,), gbase + c * 16, _i32) + iota
            eeb[r, sl] = jnp.where(idv < ETOT, ee, z16)
            d16 = idx_d[r, sl]
            idx_t[r, sl] = jnp.where(d16 < NHALF, d16,
                                     jnp.full((16,), NHALF, _i32))

    pltpu.sync_copy(eeb, ee_out.at[pl.ds(rbase, CHROWS)])

    plsc.subcore_barrier()

    @pl.loop(0, CHROWS)
    def _(j):
        pltpu.sync_copy(eeb.at[j], den_sh.at[idx_d.at[j]], add=True)

    _msg_pass(xw_hbm, idx_s, idx_t, eeb, rowsb, xatt_sh, sem, sem2, rbase)

    plsc.subcore_barrier()

    abase = sid * ASTRIPE

    @pl.when(cid == 0)
    def _():
        pltpu.sync_copy(den_sh.at[pl.ds(sid * STRIPE, STRIPE)],
                        den0_out.at[pl.ds(sid * STRIPE, STRIPE)])
        pltpu.sync_copy(xatt_sh.at[pl.ds(abase, ASTRIPE)],
                        xatt0_out.at[pl.ds(abase, ASTRIPE)])

    @pl.when(cid == 1)
    def _():
        pltpu.sync_copy(den_sh.at[pl.ds(sid * STRIPE, STRIPE)],
                        den1_out.at[pl.ds(sid * STRIPE, STRIPE)])
        pltpu.sync_copy(xatt_sh.at[pl.ds(abase, ASTRIPE)],
                        xatt1_out.at[pl.ds(abase, ASTRIPE)])


def _sc_edge_b_body(src2_hbm, dst2_hbm, xw_hbm, ee_hbm,
                    xatt0_out, xatt1_out,
                    idx_s, idx_t, eeb, rowsb, xatt_sh, sem, sem2):
    cid = lax.axis_index("c")
    sid = lax.axis_index("s")
    wid = sid * 2 + cid

    _zero_stripes(rowsb, xatt_sh, sid, None)

    rbase = wid * CHROWS
    pltpu.sync_copy(src2_hbm.at[pl.ds(rbase, CHROWS)], idx_s)
    pltpu.sync_copy(ee_hbm.at[pl.ds(rbase, CHROWS)], eeb)
    pltpu.sync_copy(dst2_hbm.at[pl.ds(rbase, CHROWS)], idx_t)

    @pl.loop(0, CHROWS)
    def _(r):
        for c in range(8):
            sl = pl.ds(c * 16, 16)
            d16 = idx_t[r, sl] - NHALF
            ok = (d16 >= 0) & (d16 < NHALF)
            idx_t[r, sl] = jnp.where(ok, d16, jnp.full((16,), NHALF, _i32))

    plsc.subcore_barrier()

    _msg_pass(xw_hbm, idx_s, idx_t, eeb, rowsb, xatt_sh, sem, sem2, rbase)

    plsc.subcore_barrier()

    abase = sid * ASTRIPE

    @pl.when(cid == 0)
    def _():
        pltpu.sync_copy(xatt_sh.at[pl.ds(abase, ASTRIPE)],
                        xatt0_out.at[pl.ds(abase, ASTRIPE)])

    @pl.when(cid == 1)
    def _():
        pltpu.sync_copy(xatt_sh.at[pl.ds(abase, ASTRIPE)],
                        xatt1_out.at[pl.ds(abase, ASTRIPE)])


@functools.cache
def _sc_edge_a():
    mesh = plsc.VectorSubcoreMesh(core_axis_name="c", subcore_axis_name="s")
    return pl.kernel(
        _sc_edge_a_body,
        out_type=[
            jax.ShapeDtypeStruct((NIROWS, 128), _f32),
            jax.ShapeDtypeStruct((NPAD,), _f32),
            jax.ShapeDtypeStruct((NPAD,), _f32),
            jax.ShapeDtypeStruct((ACCR, D), _f32),
            jax.ShapeDtypeStruct((ACCR, D), _f32),
        ],
        mesh=mesh,
        scratch_types=[
            pltpu.VMEM((CHROWS, 128), _i32),
            pltpu.VMEM((CHROWS, 128), _i32),
            pltpu.VMEM((CHROWS, 128), _i32),
            pltpu.VMEM((CHROWS, 128), _f32),
            pltpu.VMEM((CHROWS, 128), _f32),
            pltpu.VMEM((2, 128, D), _f32),
            pltpu.VMEM_SHARED((NPAD,), _f32),
            pltpu.VMEM_SHARED((ACCR, D), _f32),
            pltpu.SemaphoreType.DMA,
            pltpu.SemaphoreType.DMA,
        ],
    )


@functools.cache
def _sc_edge_b():
    mesh = plsc.VectorSubcoreMesh(core_axis_name="c", subcore_axis_name="s")
    return pl.kernel(
        _sc_edge_b_body,
        out_type=[
            jax.ShapeDtypeStruct((ACCR, D), _f32),
            jax.ShapeDtypeStruct((ACCR, D), _f32),
        ],
        mesh=mesh,
        scratch_types=[
            pltpu.VMEM((CHROWS, 128), _i32),
            pltpu.VMEM((CHROWS, 128), _i32),
            pltpu.VMEM((CHROWS, 128), _f32),
            pltpu.VMEM((2, 128, D), _f32),
            pltpu.VMEM_SHARED((ACCR, D), _f32),
            pltpu.SemaphoreType.DMA,
            pltpu.SemaphoreType.DMA,
        ],
    )


def _sc_alpha_body(ee_hbm, ivd_hbm, dst2_hbm, alpha_out, idx_d, eev, ivv, sem):
    cid = lax.axis_index("c")
    sid = lax.axis_index("s")
    wid = sid * 2 + cid
    rbase = wid * CHROWS
    pltpu.sync_copy(dst2_hbm.at[pl.ds(rbase, CHROWS)], idx_d)
    pltpu.sync_copy(ee_hbm.at[pl.ds(rbase, CHROWS)], eev)

    @pl.loop(0, CHROWS)
    def _(j):
        pltpu.async_copy(ivd_hbm.at[idx_d.at[j]], ivv.at[j], sem).wait()

    @pl.loop(0, CHROWS)
    def _(r):
        for c in range(8):
            sl = pl.ds(c * 16, 16)
            eev[r, sl] = eev[r, sl] * ivv[r, sl]

    pltpu.sync_copy(eev, alpha_out.at[pl.ds(rbase, CHROWS)])


@functools.cache
def _sc_alpha():
    mesh = plsc.VectorSubcoreMesh(core_axis_name="c", subcore_axis_name="s")
    return pl.kernel(
        _sc_alpha_body,
        out_type=jax.ShapeDtypeStruct((NIROWS, 128), _f32),
        mesh=mesh,
        scratch_types=[
            pltpu.VMEM((CHROWS, 128), _i32),
            pltpu.VMEM((CHROWS, 128), _f32),
            pltpu.VMEM((CHROWS, 128), _f32),
            pltpu.SemaphoreType.DMA,
        ],
    )


def kernel(x, We, be, bn_g, bn_b, W_gat, att_src, att_dst, b_gat,
           ln1_g, ln1_b, W1, b1, W2, b2, ln2_g, ln2_b, Wc, bc, edge_index):
    x2 = x.reshape(N, NMOD * IN_DIM)
    w_all = We.reshape(NMOD * IN_DIM, D)

    loop = jnp.arange(N, dtype=edge_index.dtype)
    pad = jnp.zeros((EP - ETOT,), dtype=edge_index.dtype)
    src2 = jnp.concatenate([edge_index[0], loop, pad]).reshape(NIROWS, 128)
    dst2 = jnp.concatenate([edge_index[1], loop, pad]).reshape(NIROWS, 128)

    h_pre, s, ss = _embed(x2, w_all, be)
    h, xw, a_s, a_d = _bn_gat(h_pre, s, ss, bn_g[None, :], bn_b[None, :],
                              W_gat[0], att_src[0].reshape(1, D),
                              att_dst[0].reshape(1, D))

    sc_a = _sc_edge_a()
    sc_b = _sc_edge_b()
    sc_alpha = _sc_alpha()
    alphas = []
    for l in range(L):
        ee, den0, den1, a0, a1 = sc_a(a_s.reshape(N), a_d.reshape(N), src2,
                                      dst2, xw)
        q0, q1 = sc_b(src2, dst2, xw, ee)
        xp0 = jnp.concatenate([a0[:NHALF], q0[:NHALF]], axis=0)
        xp1 = jnp.concatenate([a1[:NHALF], q1[:NHALF]], axis=0)
        args = (h, xp0, xp1, den0[:N, None], den1[:N, None],
                b_gat[l][None, :], ln1_g[l][None, :], ln1_b[l][None, :],
                W1[l], b1[l][None, :], W2[l], b2[l][None, :],
                ln2_g[l][None, :], ln2_b[l][None, :])
        if l < L - 1:
            h, xw, a_s, a_d, ivd = _layer_mid(
                *args, W_gat[l + 1], att_src[l + 1].reshape(1, D),
                att_dst[l + 1].reshape(1, D))
        else:
            logits, ivd = _layer_last(*args, Wc, bc[None, :])
        alpha = sc_alpha(ee, ivd.reshape(N), dst2)
        alphas.append(alpha.reshape(EP)[:ETOT].reshape(ETOT, 1))

    return (logits.reshape(N), alphas[0], alphas[1], alphas[2])


# speedup vs baseline: 10.9224x; 1.0012x over previous
import functools

import jax
import jax.numpy as jnp
from jax import lax
from jax.experimental import pallas as pl
from jax.experimental.pallas import tpu as pltpu
from jax.experimental.pallas import tpu_sc as plsc

N = 10000
E = 320000
NMOD = 3
IN_DIM = 200
D = 128
L = 3
ETOT = E + N
EP = 360448
NIROWS = EP // 128
CH = EP // 32
CHROWS = CH // 128
NHALF = N // 2
ACCR = 5120
ASTRIPE = ACCR // 16
NPAD = 10240
STRIPE = NPAD // 16
RB = 1000
NBLK = N // RB

_f32 = jnp.float32
_i32 = jnp.int32


def _embed_body(x_ref, w_ref, be_ref, h_ref, s_ref, ss_ref):
    h = jnp.dot(x_ref[...], w_ref[...], preferred_element_type=_f32)
    h = h + (be_ref[0, :] + be_ref[1, :] + be_ref[2, :])[None, :]
    h_ref[...] = h

    @pl.when(pl.program_id(0) == 0)
    def _():
        s_ref[...] = jnp.zeros_like(s_ref)
        ss_ref[...] = jnp.zeros_like(ss_ref)

    s_ref[...] += jnp.sum(h, axis=0, keepdims=True)
    ss_ref[...] += jnp.sum(h * h, axis=0, keepdims=True)


def _embed(x2, w_all, be):
    return pl.pallas_call(
        _embed_body,
        grid=(NBLK,),
        in_specs=[
            pl.BlockSpec((RB, NMOD * IN_DIM), lambda i: (i, 0)),
            pl.BlockSpec((NMOD * IN_DIM, D), lambda i: (0, 0)),
            pl.BlockSpec((NMOD, D), lambda i: (0, 0)),
        ],
        out_specs=[
            pl.BlockSpec((RB, D), lambda i: (i, 0)),
            pl.BlockSpec((1, D), lambda i: (0, 0)),
            pl.BlockSpec((1, D), lambda i: (0, 0)),
        ],
        out_shape=[
            jax.ShapeDtypeStruct((N, D), _f32),
            jax.ShapeDtypeStruct((1, D), _f32),
            jax.ShapeDtypeStruct((1, D), _f32),
        ],
    )(x2, w_all, be)


def _bn_gat_body(hp_ref, s_ref, ss_ref, g_ref, b_ref, w_ref, as_ref, ad_ref,
                 h_ref, xw_ref, aso_ref, ado_ref):
    mu = s_ref[...] / N
    var = ss_ref[...] / N - mu * mu
    h = (hp_ref[...] - mu) * jax.lax.rsqrt(var + 1e-5) * g_ref[...] + b_ref[...]
    h = jnp.maximum(h, 0.0)
    h_ref[...] = h
    xw = jnp.dot(h, w_ref[...], preferred_element_type=_f32)
    xw_ref[...] = xw
    aso_ref[...] = jnp.sum(xw * as_ref[...], axis=1, keepdims=True)
    ado_ref[...] = jnp.sum(xw * ad_ref[...], axis=1, keepdims=True)


def _bn_gat(h_pre, s, ss, bn_g, bn_b, wgat, att_s, att_d):
    row = lambda i: (i, 0)
    full = lambda i: (0, 0)
    return pl.pallas_call(
        _bn_gat_body,
        grid=(NBLK,),
        in_specs=[
            pl.BlockSpec((RB, D), row),
            pl.BlockSpec((1, D), full), pl.BlockSpec((1, D), full),
            pl.BlockSpec((1, D), full), pl.BlockSpec((1, D), full),
            pl.BlockSpec((D, D), full),
            pl.BlockSpec((1, D), full), pl.BlockSpec((1, D), full),
        ],
        out_specs=[
            pl.BlockSpec((RB, D), row), pl.BlockSpec((RB, D), row),
            pl.BlockSpec((RB, 1), row), pl.BlockSpec((RB, 1), row),
        ],
        out_shape=[
            jax.ShapeDtypeStruct((N, D), _f32),
            jax.ShapeDtypeStruct((N, D), _f32),
            jax.ShapeDtypeStruct((N, 1), _f32),
            jax.ShapeDtypeStruct((N, 1), _f32),
        ],
    )(h_pre, s, ss, bn_g, bn_b, wgat, att_s, att_d)


def _ln(t, g, b):
    mu = jnp.mean(t, axis=1, keepdims=True)
    v = jnp.mean((t - mu) ** 2, axis=1, keepdims=True)
    return (t - mu) * jax.lax.rsqrt(v + 1e-5) * g + b


def _ffn_block(h, xp0_ref, xp1_ref, d0_ref, d1_ref, bg_ref, l1g_ref, l1b_ref,
               w1_ref, b1_ref, w2_ref, b2_ref, l2g_ref, l2b_ref):
    ivd = 1.0 / (d0_ref[:, 0] + d1_ref[:, 0] + 1e-16)
    xatt = (xp0_ref[...] + xp1_ref[...]) * ivd[:, None] + bg_ref[...]
    h1 = _ln(h + xatt, l1g_ref[...], l1b_ref[...])
    f = jnp.dot(jnp.maximum(
        jnp.dot(h1, w1_ref[...], preferred_element_type=_f32) + b1_ref[...],
        0.0), w2_ref[...], preferred_element_type=_f32) + b2_ref[...]
    return _ln(h1 + f, l2g_ref[...], l2b_ref[...]), ivd


def _layer_mid_body(h_ref, xp0_ref, xp1_ref, d0_ref, d1_ref, bg_ref,
                    l1g_ref, l1b_ref, w1_ref, b1_ref, w2_ref, b2_ref,
                    l2g_ref, l2b_ref, w_ref, as_ref, ad_ref,
                    hn_ref, xw_ref, aso_ref, ado_ref, ivd_ref):
    hn, ivd = _ffn_block(h_ref[...], xp0_ref, xp1_ref, d0_ref, d1_ref, bg_ref,
                         l1g_ref, l1b_ref, w1_ref, b1_ref, w2_ref, b2_ref,
                         l2g_ref, l2b_ref)
    hn_ref[...] = hn
    ivd_ref[...] = ivd[:, None]
    xw = jnp.dot(hn, w_ref[...], preferred_element_type=_f32)
    xw_ref[...] = xw
    aso_ref[...] = jnp.sum(xw * as_ref[...], axis=1, keepdims=True)
    ado_ref[...] = jnp.sum(xw * ad_ref[...], axis=1, keepdims=True)


def _layer_mid(h, xp0, xp1, d0, d1, bg, l1g, l1b, w1, b1, w2, b2, l2g, l2b,
               wgat, att_s, att_d):
    row = lambda i: (i, 0)
    full = lambda i: (0, 0)
    return pl.pallas_call(
        _layer_mid_body,
        grid=(NBLK,),
        in_specs=[
            pl.BlockSpec((RB, D), row), pl.BlockSpec((RB, D), row),
            pl.BlockSpec((RB, D), row),
            pl.BlockSpec((RB, 1), row), pl.BlockSpec((RB, 1), row),
            pl.BlockSpec((1, D), full), pl.BlockSpec((1, D), full),
            pl.BlockSpec((1, D), full),
            pl.BlockSpec((D, D), full), pl.BlockSpec((1, D), full),
            pl.BlockSpec((D, D), full), pl.BlockSpec((1, D), full),
            pl.BlockSpec((1, D), full), pl.BlockSpec((1, D), full),
            pl.BlockSpec((D, D), full), pl.BlockSpec((1, D), full),
            pl.BlockSpec((1, D), full),
        ],
        out_specs=[
            pl.BlockSpec((RB, D), row), pl.BlockSpec((RB, D), row),
            pl.BlockSpec((RB, 1), row), pl.BlockSpec((RB, 1), row),
            pl.BlockSpec((RB, 1), row),
        ],
        out_shape=[
            jax.ShapeDtypeStruct((N, D), _f32),
            jax.ShapeDtypeStruct((N, D), _f32),
            jax.ShapeDtypeStruct((N, 1), _f32),
            jax.ShapeDtypeStruct((N, 1), _f32),
            jax.ShapeDtypeStruct((N, 1), _f32),
        ],
    )(h, xp0, xp1, d0, d1, bg, l1g, l1b, w1, b1, w2, b2, l2g, l2b,
      wgat, att_s, att_d)


def _layer_last_body(h_ref, xp0_ref, xp1_ref, d0_ref, d1_ref, bg_ref,
                     l1g_ref, l1b_ref, w1_ref, b1_ref, w2_ref, b2_ref,
                     l2g_ref, l2b_ref, wc_ref, bc_ref, out_ref, ivd_ref):
    hn, ivd = _ffn_block(h_ref[...], xp0_ref, xp1_ref, d0_ref, d1_ref, bg_ref,
                         l1g_ref, l1b_ref, w1_ref, b1_ref, w2_ref, b2_ref,
                         l2g_ref, l2b_ref)
    ivd_ref[...] = ivd[:, None]
    out_ref[...] = jnp.dot(hn, wc_ref[...], preferred_element_type=_f32) + bc_ref[...]


def _layer_last(h, xp0, xp1, d0, d1, bg, l1g, l1b, w1, b1, w2, b2, l2g, l2b,
                wc, bc):
    row = lambda i: (i, 0)
    full = lambda i: (0, 0)
    return pl.pallas_call(
        _layer_last_body,
        grid=(NBLK,),
        in_specs=[
            pl.BlockSpec((RB, D), row), pl.BlockSpec((RB, D), row),
            pl.BlockSpec((RB, D), row),
            pl.BlockSpec((RB, 1), row), pl.BlockSpec((RB, 1), row),
            pl.BlockSpec((1, D), full), pl.BlockSpec((1, D), full),
            pl.BlockSpec((1, D), full),
            pl.BlockSpec((D, D), full), pl.BlockSpec((1, D), full),
            pl.BlockSpec((D, D), full), pl.BlockSpec((1, D), full),
            pl.BlockSpec((1, D), full), pl.BlockSpec((1, D), full),
            pl.BlockSpec((D, 1), full), pl.BlockSpec((1, 1), full),
        ],
        out_specs=[
            pl.BlockSpec((RB, 1), row),
            pl.BlockSpec((RB, 1), row),
        ],
        out_shape=[
            jax.ShapeDtypeStruct((N, 1), _f32),
            jax.ShapeDtypeStruct((N, 1), _f32),
        ],
    )(h, xp0, xp1, d0, d1, bg, l1g, l1b, w1, b1, w2, b2, l2g, l2b, wc, bc)


def _scale_rows(rowsb, b, eeb, j):
    @pl.loop(0, 8)
    def _(g):
        a16 = eeb[j, pl.ds(g * 16, 16)]
        for kk in range(16):
            av = a16[jnp.full((16,), kk, _i32)]
            r = g * 16 + kk
            for c in range(8):
                sl = pl.ds(c * 16, 16)
                rowsb[b, r, sl] = rowsb[b, r, sl] * av


def _msg_pass(xw_hbm, idx_s, idx_t, eeb, rowsb, xatt_sh, sem, sem2, rbase):
    @pl.loop(0, CHROWS, step=2)
    def _(j0):
        live0 = rbase * 128 + j0 * 128 < ETOT
        live1 = rbase * 128 + (j0 + 1) * 128 < ETOT

        @pl.when(live0)
        def _():
            c0 = pltpu.async_copy(xw_hbm.at[idx_s.at[j0]], rowsb.at[0], sem)

            @pl.when(live1)
            def _():
                pltpu.async_copy(xw_hbm.at[idx_s.at[j0 + 1]], rowsb.at[1],
                                 sem2)

            c0.wait()
            _scale_rows(rowsb, 0, eeb, j0)
            pltpu.sync_copy(rowsb.at[0], xatt_sh.at[idx_t.at[j0]], add=True)

            @pl.when(live1)
            def _():
                pltpu.make_async_copy(xw_hbm.at[idx_s.at[j0 + 1]],
                                      rowsb.at[1], sem2).wait()
                _scale_rows(rowsb, 1, eeb, j0 + 1)
                pltpu.sync_copy(rowsb.at[1], xatt_sh.at[idx_t.at[j0 + 1]],
                                add=True)


def _zero_stripes(rowsb, xatt_sh, sid, den_sh):
    z16 = jnp.zeros((16,), _f32)

    @pl.loop(0, 128)
    def _(i):
        for c in range(8):
            rowsb[0, i, pl.ds(c * 16, 16)] = z16

    abase = sid * ASTRIPE
    pltpu.sync_copy(rowsb.at[0], xatt_sh.at[pl.ds(abase, 128)])
    pltpu.sync_copy(rowsb.at[0], xatt_sh.at[pl.ds(abase + 128, 128)])
    pltpu.sync_copy(rowsb.at[0, pl.ds(0, 64)],
                    xatt_sh.at[pl.ds(abase + 256, 64)])
    if den_sh is not None:
        for k in range(STRIPE // 128):
            pltpu.sync_copy(rowsb.at[0, k],
                            den_sh.at[pl.ds(sid * STRIPE + k * 128, 128)])


def _sc_edge_a_body(as_hbm, ad_hbm, src2_hbm, dst2_hbm, xw_hbm,
                    ee_out, den0_out, den1_out, xatt0_out, xatt1_out,
                    idx_s, idx_d, idx_t, ase, eeb, rowsb,
                    den_sh, xatt_sh, sem, sem2):
    cid = lax.axis_index("c")
    sid = lax.axis_index("s")
    wid = sid * 2 + cid
    iota = lax.iota(_i32, 16)
    z16 = jnp.zeros((16,), _f32)

    _zero_stripes(rowsb, xatt_sh, sid, den_sh)

    rbase = wid * CHROWS
    pltpu.sync_copy(src2_hbm.at[pl.ds(rbase, CHROWS)], idx_s)
    pltpu.sync_copy(dst2_hbm.at[pl.ds(rbase, CHROWS)], idx_d)

    @pl.loop(0, CHROWS, step=8)
    def _(j0):
        cs = []
        for b in range(8):
            cs.append(pltpu.async_copy(as_hbm.at[idx_s.at[j0 + b]],
                                       ase.at[j0 + b], sem))
            cs.append(pltpu.async_copy(ad_hbm.at[idx_d.at[j0 + b]],
                                       eeb.at[j0 + b], sem2))
        for c in cs:
            c.wait()

    @pl.loop(0, CHROWS)
    def _(r):
        gbase = rbase * 128 + r * 128
        for c in range(8):
            sl = pl.ds(c * 16, 16)
            ev = ase[r, sl] + eeb[r, sl]
            ev = jnp.where(ev > 0, ev, 0.2 * ev)
            ee = jnp.exp(ev)
            idv = jnp.full((16,), gbase + c * 16, _i32) + iota
            eeb[r, sl] = jnp.where(idv < ETOT, ee, z16)
            d16 = idx_d[r, sl]
            idx_t[r, sl] = jnp.where(d16 < NHALF, d16,
                                     jnp.full((16,), NHALF, _i32))

    pltpu.sync_copy(eeb, ee_out.at[pl.ds(rbase, CHROWS)])

    plsc.subcore_barrier()

    @pl.loop(0, CHROWS)
    def _(j):
        pltpu.sync_copy(eeb.at[j], den_sh.at[idx_d.at[j]], add=True)

    _msg_pass(xw_hbm, idx_s, idx_t, eeb, rowsb, xatt_sh, sem, sem2, rbase)

    plsc.subcore_barrier()

    abase = sid * ASTRIPE

    @pl.when(cid == 0)
    def _():
        pltpu.sync_copy(den_sh.at[pl.ds(sid * STRIPE, STRIPE)],
                        den0_out.at[pl.ds(sid * STRIPE, STRIPE)])
        pltpu.sync_copy(xatt_sh.at[pl.ds(abase, ASTRIPE)],
                        xatt0_out.at[pl.ds(abase, ASTRIPE)])

    @pl.when(cid == 1)
    def _():
        pltpu.sync_copy(den_sh.at[pl.ds(sid * STRIPE, STRIPE)],
                        den1_out.at[pl.ds(sid * STRIPE, STRIPE)])
        pltpu.sync_copy(xatt_sh.at[pl.ds(abase, ASTRIPE)],
                        xatt1_out.at[pl.ds(abase, ASTRIPE)])


def _sc_edge_b_body(src2_hbm, dst2_hbm, xw_hbm, ee_hbm,
                    xatt0_out, xatt1_out,
                    idx_s, idx_t, eeb, rowsb, xatt_sh, sem, sem2):
    cid = lax.axis_index("c")
    sid = lax.axis_index("s")
    wid = sid * 2 + cid

    _zero_stripes(rowsb, xatt_sh, sid, None)

    rbase = wid * CHROWS
    pltpu.sync_copy(src2_hbm.at[pl.ds(rbase, CHROWS)], idx_s)
    pltpu.sync_copy(ee_hbm.at[pl.ds(rbase, CHROWS)], eeb)
    pltpu.sync_copy(dst2_hbm.at[pl.ds(rbase, CHROWS)], idx_t)

    @pl.loop(0, CHROWS)
    def _(r):
        for c in range(8):
            sl = pl.ds(c * 16, 16)
            d16 = idx_t[r, sl] - NHALF
            ok = (d16 >= 0) & (d16 < NHALF)
            idx_t[r, sl] = jnp.where(ok, d16, jnp.full((16,), NHALF, _i32))

    plsc.subcore_barrier()

    _msg_pass(xw_hbm, idx_s, idx_t, eeb, rowsb, xatt_sh, sem, sem2, rbase)

    plsc.subcore_barrier()

    abase = sid * ASTRIPE

    @pl.when(cid == 0)
    def _():
        pltpu.sync_copy(xatt_sh.at[pl.ds(abase, ASTRIPE)],
                        xatt0_out.at[pl.ds(abase, ASTRIPE)])

    @pl.when(cid == 1)
    def _():
        pltpu.sync_copy(xatt_sh.at[pl.ds(abase, ASTRIPE)],
                        xatt1_out.at[pl.ds(abase, ASTRIPE)])


@functools.cache
def _sc_edge_a():
    mesh = plsc.VectorSubcoreMesh(core_axis_name="c", subcore_axis_name="s")
    return pl.kernel(
        _sc_edge_a_body,
        out_type=[
            jax.ShapeDtypeStruct((NIROWS, 128), _f32),
            jax.ShapeDtypeStruct((NPAD,), _f32),
            jax.ShapeDtypeStruct((NPAD,), _f32),
            jax.ShapeDtypeStruct((ACCR, D), _f32),
            jax.ShapeDtypeStruct((ACCR, D), _f32),
        ],
        mesh=mesh,
        scratch_types=[
            pltpu.VMEM((CHROWS, 128), _i32),
            pltpu.VMEM((CHROWS, 128), _i32),
            pltpu.VMEM((CHROWS, 128), _i32),
            pltpu.VMEM((CHROWS, 128), _f32),
            pltpu.VMEM((CHROWS, 128), _f32),
            pltpu.VMEM((2, 128, D), _f32),
            pltpu.VMEM_SHARED((NPAD,), _f32),
            pltpu.VMEM_SHARED((ACCR, D), _f32),
            pltpu.SemaphoreType.DMA,
            pltpu.SemaphoreType.DMA,
        ],
    )


@functools.cache
def _sc_edge_b():
    mesh = plsc.VectorSubcoreMesh(core_axis_name="c", subcore_axis_name="s")
    return pl.kernel(
        _sc_edge_b_body,
        out_type=[
            jax.ShapeDtypeStruct((ACCR, D), _f32),
            jax.ShapeDtypeStruct((ACCR, D), _f32),
        ],
        mesh=mesh,
        scratch_types=[
            pltpu.VMEM((CHROWS, 128), _i32),
            pltpu.VMEM((CHROWS, 128), _i32),
            pltpu.VMEM((CHROWS, 128), _f32),
            pltpu.VMEM((2, 128, D), _f32),
            pltpu.VMEM_SHARED((ACCR, D), _f32),
            pltpu.SemaphoreType.DMA,
            pltpu.SemaphoreType.DMA,
        ],
    )


def _sc_alpha_body(ee_hbm, ivd_hbm, dst2_hbm, alpha_out, idx_d, eev, ivv, sem):
    cid = lax.axis_index("c")
    sid = lax.axis_index("s")
    wid = sid * 2 + cid
    rbase = wid * CHROWS
    pltpu.sync_copy(dst2_hbm.at[pl.ds(rbase, CHROWS)], idx_d)
    pltpu.sync_copy(ee_hbm.at[pl.ds(rbase, CHROWS)], eev)

    @pl.loop(0, CHROWS, step=8)
    def _(j0):
        cs = []
        for b in range(8):
            cs.append(pltpu.async_copy(ivd_hbm.at[idx_d.at[j0 + b]],
                                       ivv.at[j0 + b], sem))
        for c in cs:
            c.wait()

    @pl.loop(0, CHROWS)
    def _(r):
        for c in range(8):
            sl = pl.ds(c * 16, 16)
            eev[r, sl] = eev[r, sl] * ivv[r, sl]

    pltpu.sync_copy(eev, alpha_out.at[pl.ds(rbase, CHROWS)])


@functools.cache
def _sc_alpha():
    mesh = plsc.VectorSubcoreMesh(core_axis_name="c", subcore_axis_name="s")
    return pl.kernel(
        _sc_alpha_body,
        out_type=jax.ShapeDtypeStruct((NIROWS, 128), _f32),
        mesh=mesh,
        scratch_types=[
            pltpu.VMEM((CHROWS, 128), _i32),
            pltpu.VMEM((CHROWS, 128), _f32),
            pltpu.VMEM((CHROWS, 128), _f32),
            pltpu.SemaphoreType.DMA,
        ],
    )


def kernel(x, We, be, bn_g, bn_b, W_gat, att_src, att_dst, b_gat,
           ln1_g, ln1_b, W1, b1, W2, b2, ln2_g, ln2_b, Wc, bc, edge_index):
    x2 = x.reshape(N, NMOD * IN_DIM)
    w_all = We.reshape(NMOD * IN_DIM, D)

    loop = jnp.arange(N, dtype=edge_index.dtype)
    pad = jnp.zeros((EP - ETOT,), dtype=edge_index.dtype)
    src2 = jnp.concatenate([edge_index[0], loop, pad]).reshape(NIROWS, 128)
    dst2 = jnp.concatenate([edge_index[1], loop, pad]).reshape(NIROWS, 128)

    h_pre, s, ss = _embed(x2, w_all, be)
    h, xw, a_s, a_d = _bn_gat(h_pre, s, ss, bn_g[None, :], bn_b[None, :],
                              W_gat[0], att_src[0].reshape(1, D),
                              att_dst[0].reshape(1, D))

    sc_a = _sc_edge_a()
    sc_b = _sc_edge_b()
    sc_alpha = _sc_alpha()
    alphas = []
    for l in range(L):
        ee, den0, den1, a0, a1 = sc_a(a_s.reshape(N), a_d.reshape(N), src2,
                                      dst2, xw)
        q0, q1 = sc_b(src2, dst2, xw, ee)
        xp0 = jnp.concatenate([a0[:NHALF], q0[:NHALF]], axis=0)
        xp1 = jnp.concatenate([a1[:NHALF], q1[:NHALF]], axis=0)
        args = (h, xp0, xp1, den0[:N, None], den1[:N, None],
                b_gat[l][None, :], ln1_g[l][None, :], ln1_b[l][None, :],
                W1[l], b1[l][None, :], W2[l], b2[l][None, :],
                ln2_g[l][None, :], ln2_b[l][None, :])
        if l < L - 1:
            h, xw, a_s, a_d, ivd = _layer_mid(
                *args, W_gat[l + 1], att_src[l + 1].reshape(1, D),
                att_dst[l + 1].reshape(1, D))
        else:
            logits, ivd = _layer_last(*args, Wc, bc[None, :])
        alpha = sc_alpha(ee, ivd.reshape(N), dst2)
        alphas.append(alpha.reshape(EP)[:ETOT].reshape(ETOT, 1))

    return (logits.reshape(N), alphas[0], alphas[1], alphas[2])


# speedup vs baseline: 11.2664x; 1.0315x over previous
import functools

import jax
import jax.numpy as jnp
from jax import lax
from jax.experimental import pallas as pl
from jax.experimental.pallas import tpu as pltpu
from jax.experimental.pallas import tpu_sc as plsc

N = 10000
E = 320000
NMOD = 3
IN_DIM = 200
D = 128
L = 3
ETOT = E + N
EP = 360448
NIROWS = EP // 128
CH = EP // 32
CHROWS = CH // 128
NHALF = N // 2
ACCR = 5120
ASTRIPE = ACCR // 16
NPAD = 10240
STRIPE = NPAD // 16
RB = 1000
NBLK = N // RB

_f32 = jnp.float32
_i32 = jnp.int32


def _embed_body(x_ref, w_ref, be_ref, h_ref, s_ref, ss_ref):
    h = jnp.dot(x_ref[...], w_ref[...], preferred_element_type=_f32)
    h = h + (be_ref[0, :] + be_ref[1, :] + be_ref[2, :])[None, :]
    h_ref[...] = h

    @pl.when(pl.program_id(0) == 0)
    def _():
        s_ref[...] = jnp.zeros_like(s_ref)
        ss_ref[...] = jnp.zeros_like(ss_ref)

    s_ref[...] += jnp.sum(h, axis=0, keepdims=True)
    ss_ref[...] += jnp.sum(h * h, axis=0, keepdims=True)


def _embed(x2, w_all, be):
    return pl.pallas_call(
        _embed_body,
        grid=(NBLK,),
        in_specs=[
            pl.BlockSpec((RB, NMOD * IN_DIM), lambda i: (i, 0)),
            pl.BlockSpec((NMOD * IN_DIM, D), lambda i: (0, 0)),
            pl.BlockSpec((NMOD, D), lambda i: (0, 0)),
        ],
        out_specs=[
            pl.BlockSpec((RB, D), lambda i: (i, 0)),
            pl.BlockSpec((1, D), lambda i: (0, 0)),
            pl.BlockSpec((1, D), lambda i: (0, 0)),
        ],
        out_shape=[
            jax.ShapeDtypeStruct((N, D), _f32),
            jax.ShapeDtypeStruct((1, D), _f32),
            jax.ShapeDtypeStruct((1, D), _f32),
        ],
    )(x2, w_all, be)


def _bn_gat_body(hp_ref, s_ref, ss_ref, g_ref, b_ref, w_ref, as_ref, ad_ref,
                 h_ref, xw_ref, aso_ref, ado_ref):
    mu = s_ref[...] / N
    var = ss_ref[...] / N - mu * mu
    h = (hp_ref[...] - mu) * jax.lax.rsqrt(var + 1e-5) * g_ref[...] + b_ref[...]
    h = jnp.maximum(h, 0.0)
    h_ref[...] = h
    xw = jnp.dot(h, w_ref[...], preferred_element_type=_f32)
    xw_ref[...] = xw
    aso_ref[...] = jnp.sum(xw * as_ref[...], axis=1, keepdims=True)
    ado_ref[...] = jnp.sum(xw * ad_ref[...], axis=1, keepdims=True)


def _bn_gat(h_pre, s, ss, bn_g, bn_b, wgat, att_s, att_d):
    row = lambda i: (i, 0)
    full = lambda i: (0, 0)
    return pl.pallas_call(
        _bn_gat_body,
        grid=(NBLK,),
        in_specs=[
            pl.BlockSpec((RB, D), row),
            pl.BlockSpec((1, D), full), pl.BlockSpec((1, D), full),
            pl.BlockSpec((1, D), full), pl.BlockSpec((1, D), full),
            pl.BlockSpec((D, D), full),
            pl.BlockSpec((1, D), full), pl.BlockSpec((1, D), full),
        ],
        out_specs=[
            pl.BlockSpec((RB, D), row), pl.BlockSpec((RB, D), row),
            pl.BlockSpec((RB, 1), row), pl.BlockSpec((RB, 1), row),
        ],
        out_shape=[
            jax.ShapeDtypeStruct((N, D), _f32),
            jax.ShapeDtypeStruct((N, D), _f32),
            jax.ShapeDtypeStruct((N, 1), _f32),
            jax.ShapeDtypeStruct((N, 1), _f32),
        ],
    )(h_pre, s, ss, bn_g, bn_b, wgat, att_s, att_d)


def _ln(t, g, b):
    mu = jnp.mean(t, axis=1, keepdims=True)
    v = jnp.mean((t - mu) ** 2, axis=1, keepdims=True)
    return (t - mu) * jax.lax.rsqrt(v + 1e-5) * g + b


def _ffn_block(h, xp0_ref, xp1_ref, d0_ref, d1_ref, bg_ref, l1g_ref, l1b_ref,
               w1_ref, b1_ref, w2_ref, b2_ref, l2g_ref, l2b_ref):
    ivd = 1.0 / (d0_ref[:, 0] + d1_ref[:, 0] + 1e-16)
    xatt = (xp0_ref[...] + xp1_ref[...]) * ivd[:, None] + bg_ref[...]
    h1 = _ln(h + xatt, l1g_ref[...], l1b_ref[...])
    f = jnp.dot(jnp.maximum(
        jnp.dot(h1, w1_ref[...], preferred_element_type=_f32) + b1_ref[...],
        0.0), w2_ref[...], preferred_element_type=_f32) + b2_ref[...]
    return _ln(h1 + f, l2g_ref[...], l2b_ref[...]), ivd


def _layer_mid_body(h_ref, xp0_ref, xp1_ref, d0_ref, d1_ref, bg_ref,
                    l1g_ref, l1b_ref, w1_ref, b1_ref, w2_ref, b2_ref,
                    l2g_ref, l2b_ref, w_ref, as_ref, ad_ref,
                    hn_ref, xw_ref, aso_ref, ado_ref, ivd_ref):
    hn, ivd = _ffn_block(h_ref[...], xp0_ref, xp1_ref, d0_ref, d1_ref, bg_ref,
                         l1g_ref, l1b_ref, w1_ref, b1_ref, w2_ref, b2_ref,
                         l2g_ref, l2b_ref)
    hn_ref[...] = hn
    ivd_ref[...] = ivd[:, None]
    xw = jnp.dot(hn, w_ref[...], preferred_element_type=_f32)
    xw_ref[...] = xw
    aso_ref[...] = jnp.sum(xw * as_ref[...], axis=1, keepdims=True)
    ado_ref[...] = jnp.sum(xw * ad_ref[...], axis=1, keepdims=True)


def _layer_mid(h, xp0, xp1, d0, d1, bg, l1g, l1b, w1, b1, w2, b2, l2g, l2b,
               wgat, att_s, att_d):
    row = lambda i: (i, 0)
    full = lambda i: (0, 0)
    return pl.pallas_call(
        _layer_mid_body,
        grid=(NBLK,),
        in_specs=[
            pl.BlockSpec((RB, D), row), pl.BlockSpec((RB, D), row),
            pl.BlockSpec((RB, D), row),
            pl.BlockSpec((RB, 1), row), pl.BlockSpec((RB, 1), row),
            pl.BlockSpec((1, D), full), pl.BlockSpec((1, D), full),
            pl.BlockSpec((1, D), full),
            pl.BlockSpec((D, D), full), pl.BlockSpec((1, D), full),
            pl.BlockSpec((D, D), full), pl.BlockSpec((1, D), full),
            pl.BlockSpec((1, D), full), pl.BlockSpec((1, D), full),
            pl.BlockSpec((D, D), full), pl.BlockSpec((1, D), full),
            pl.BlockSpec((1, D), full),
        ],
        out_specs=[
            pl.BlockSpec((RB, D), row), pl.BlockSpec((RB, D), row),
            pl.BlockSpec((RB, 1), row), pl.BlockSpec((RB, 1), row),
            pl.BlockSpec((RB, 1), row),
        ],
        out_shape=[
            jax.ShapeDtypeStruct((N, D), _f32),
            jax.ShapeDtypeStruct((N, D), _f32),
            jax.ShapeDtypeStruct((N, 1), _f32),
            jax.ShapeDtypeStruct((N, 1), _f32),
            jax.ShapeDtypeStruct((N, 1), _f32),
        ],
    )(h, xp0, xp1, d0, d1, bg, l1g, l1b, w1, b1, w2, b2, l2g, l2b,
      wgat, att_s, att_d)


def _layer_last_body(h_ref, xp0_ref, xp1_ref, d0_ref, d1_ref, bg_ref,
                     l1g_ref, l1b_ref, w1_ref, b1_ref, w2_ref, b2_ref,
                     l2g_ref, l2b_ref, wc_ref, bc_ref, out_ref, ivd_ref):
    hn, ivd = _ffn_block(h_ref[...], xp0_ref, xp1_ref, d0_ref, d1_ref, bg_ref,
                         l1g_ref, l1b_ref, w1_ref, b1_ref, w2_ref, b2_ref,
                         l2g_ref, l2b_ref)
    ivd_ref[...] = ivd[:, None]
    out_ref[...] = jnp.dot(hn, wc_ref[...], preferred_element_type=_f32) + bc_ref[...]


def _layer_last(h, xp0, xp1, d0, d1, bg, l1g, l1b, w1, b1, w2, b2, l2g, l2b,
                wc, bc):
    row = lambda i: (i, 0)
    full = lambda i: (0, 0)
    return pl.pallas_call(
        _layer_last_body,
        grid=(NBLK,),
        in_specs=[
            pl.BlockSpec((RB, D), row), pl.BlockSpec((RB, D), row),
            pl.BlockSpec((RB, D), row),
            pl.BlockSpec((RB, 1), row), pl.BlockSpec((RB, 1), row),
            pl.BlockSpec((1, D), full), pl.BlockSpec((1, D), full),
            pl.BlockSpec((1, D), full),
            pl.BlockSpec((D, D), full), pl.BlockSpec((1, D), full),
            pl.BlockSpec((D, D), full), pl.BlockSpec((1, D), full),
            pl.BlockSpec((1, D), full), pl.BlockSpec((1, D), full),
            pl.BlockSpec((D, 1), full), pl.BlockSpec((1, 1), full),
        ],
        out_specs=[
            pl.BlockSpec((RB, 1), row),
            pl.BlockSpec((RB, 1), row),
        ],
        out_shape=[
            jax.ShapeDtypeStruct((N, 1), _f32),
            jax.ShapeDtypeStruct((N, 1), _f32),
        ],
    )(h, xp0, xp1, d0, d1, bg, l1g, l1b, w1, b1, w2, b2, l2g, l2b, wc, bc)


def _scale_rows(rowsb, b, eeb, j):
    @pl.loop(0, 8)
    def _(g):
        a16 = eeb[j, pl.ds(g * 16, 16)]
        for kk in range(16):
            av = a16[jnp.full((16,), kk, _i32)]
            r = g * 16 + kk
            for c in range(8):
                sl = pl.ds(c * 16, 16)
                rowsb[b, r, sl] = rowsb[b, r, sl] * av


def _msg_pass(xw_hbm, idx_s, idx_t, eeb, rowsb, xatt_sh, sem, sem2, sem3,
              rbase):
    @pl.loop(0, CHROWS, step=2)
    def _(j0):
        live0 = rbase * 128 + j0 * 128 < ETOT
        live1 = rbase * 128 + (j0 + 1) * 128 < ETOT

        @pl.when(live0)
        def _():
            c0 = pltpu.async_copy(xw_hbm.at[idx_s.at[j0]], rowsb.at[0], sem)

            @pl.when(live1)
            def _():
                pltpu.async_copy(xw_hbm.at[idx_s.at[j0 + 1]], rowsb.at[1],
                                 sem2)

            c0.wait()
            _scale_rows(rowsb, 0, eeb, j0)
            s0 = pltpu.async_copy(rowsb.at[0], xatt_sh.at[idx_t.at[j0]],
                                  sem3, add=True)

            @pl.when(live1)
            def _():
                pltpu.make_async_copy(xw_hbm.at[idx_s.at[j0 + 1]],
                                      rowsb.at[1], sem2).wait()
                _scale_rows(rowsb, 1, eeb, j0 + 1)
                pltpu.sync_copy(rowsb.at[1], xatt_sh.at[idx_t.at[j0 + 1]],
                                add=True)

            s0.wait()


def _zero_stripes(rowsb, xatt_sh, sid, den_sh):
    z16 = jnp.zeros((16,), _f32)

    @pl.loop(0, 128)
    def _(i):
        for c in range(8):
            rowsb[0, i, pl.ds(c * 16, 16)] = z16

    abase = sid * ASTRIPE
    pltpu.sync_copy(rowsb.at[0], xatt_sh.at[pl.ds(abase, 128)])
    pltpu.sync_copy(rowsb.at[0], xatt_sh.at[pl.ds(abase + 128, 128)])
    pltpu.sync_copy(rowsb.at[0, pl.ds(0, 64)],
                    xatt_sh.at[pl.ds(abase + 256, 64)])
    if den_sh is not None:
        for k in range(STRIPE // 128):
            pltpu.sync_copy(rowsb.at[0, k],
                            den_sh.at[pl.ds(sid * STRIPE + k * 128, 128)])


def _sc_edge_a_body(as_hbm, ad_hbm, src2_hbm, dst2_hbm, xw_hbm,
                    ee_out, den0_out, den1_out, xatt0_out, xatt1_out,
                    idx_s, idx_d, idx_t, ase, eeb, rowsb,
                    den_sh, xatt_sh, sem, sem2, sem3):
    cid = lax.axis_index("c")
    sid = lax.axis_index("s")
    wid = sid * 2 + cid
    iota = lax.iota(_i32, 16)
    z16 = jnp.zeros((16,), _f32)

    _zero_stripes(rowsb, xatt_sh, sid, den_sh)

    rbase = wid * CHROWS
    pltpu.sync_copy(src2_hbm.at[pl.ds(rbase, CHROWS)], idx_s)
    pltpu.sync_copy(dst2_hbm.at[pl.ds(rbase, CHROWS)], idx_d)

    @pl.loop(0, CHROWS, step=8)
    def _(j0):
        cs = []
        for b in range(8):
            cs.append(pltpu.async_copy(as_hbm.at[idx_s.at[j0 + b]],
                                       ase.at[j0 + b], sem))
            cs.append(pltpu.async_copy(ad_hbm.at[idx_d.at[j0 + b]],
                                       eeb.at[j0 + b], sem2))
        for c in cs:
            c.wait()

    @pl.loop(0, CHROWS)
    def _(r):
        gbase = rbase * 128 + r * 128
        for c in range(8):
            sl = pl.ds(c * 16, 16)
            ev = ase[r, sl] + eeb[r, sl]
            ev = jnp.where(ev > 0, ev, 0.2 * ev)
            ee = jnp.exp(ev)
            idv = jnp.full((16,), gbase + c * 16, _i32) + iota
            eeb[r, sl] = jnp.where(idv < ETOT, ee, z16)
            d16 = idx_d[r, sl]
            idx_t[r, sl] = jnp.where(d16 < NHALF, d16,
                                     jnp.full((16,), NHALF, _i32))

    pltpu.sync_copy(eeb, ee_out.at[pl.ds(rbase, CHROWS)])

    plsc.subcore_barrier()

    @pl.loop(0, CHROWS, step=8)
    def _(j0):
        cs = []
        for b in range(8):
            cs.append(pltpu.async_copy(eeb.at[j0 + b],
                                       den_sh.at[idx_d.at[j0 + b]], sem,
                                       add=True))
        for c in cs:
            c.wait()

    _msg_pass(xw_hbm, idx_s, idx_t, eeb, rowsb, xatt_sh, sem, sem2, sem3,
              rbase)

    plsc.subcore_barrier()

    abase = sid * ASTRIPE

    @pl.when(cid == 0)
    def _():
        pltpu.sync_copy(den_sh.at[pl.ds(sid * STRIPE, STRIPE)],
                        den0_out.at[pl.ds(sid * STRIPE, STRIPE)])
        pltpu.sync_copy(xatt_sh.at[pl.ds(abase, ASTRIPE)],
                        xatt0_out.at[pl.ds(abase, ASTRIPE)])

    @pl.when(cid == 1)
    def _():
        pltpu.sync_copy(den_sh.at[pl.ds(sid * STRIPE, STRIPE)],
                        den1_out.at[pl.ds(sid * STRIPE, STRIPE)])
        pltpu.sync_copy(xatt_sh.at[pl.ds(abase, ASTRIPE)],
                        xatt1_out.at[pl.ds(abase, ASTRIPE)])


def _sc_edge_b_body(src2_hbm, dst2_hbm, xw_hbm, ee_hbm,
                    xatt0_out, xatt1_out,
                    idx_s, idx_t, eeb, rowsb, xatt_sh, sem, sem2, sem3):
    cid = lax.axis_index("c")
    sid = lax.axis_index("s")
    wid = sid * 2 + cid

    _zero_stripes(rowsb, xatt_sh, sid, None)

    rbase = wid * CHROWS
    pltpu.sync_copy(src2_hbm.at[pl.ds(rbase, CHROWS)], idx_s)
    pltpu.sync_copy(ee_hbm.at[pl.ds(rbase, CHROWS)], eeb)
    pltpu.sync_copy(dst2_hbm.at[pl.ds(rbase, CHROWS)], idx_t)

    @pl.loop(0, CHROWS)
    def _(r):
        for c in range(8):
            sl = pl.ds(c * 16, 16)
            d16 = idx_t[r, sl] - NHALF
            ok = (d16 >= 0) & (d16 < NHALF)
            idx_t[r, sl] = jnp.where(ok, d16, jnp.full((16,), NHALF, _i32))

    plsc.subcore_barrier()

    _msg_pass(xw_hbm, idx_s, idx_t, eeb, rowsb, xatt_sh, sem, sem2, sem3,
              rbase)

    plsc.subcore_barrier()

    abase = sid * ASTRIPE

    @pl.when(cid == 0)
    def _():
        pltpu.sync_copy(xatt_sh.at[pl.ds(abase, ASTRIPE)],
                        xatt0_out.at[pl.ds(abase, ASTRIPE)])

    @pl.when(cid == 1)
    def _():
        pltpu.sync_copy(xatt_sh.at[pl.ds(abase, ASTRIPE)],
                        xatt1_out.at[pl.ds(abase, ASTRIPE)])


@functools.cache
def _sc_edge_a():
    mesh = plsc.VectorSubcoreMesh(core_axis_name="c", subcore_axis_name="s")
    return pl.kernel(
        _sc_edge_a_body,
        out_type=[
            jax.ShapeDtypeStruct((NIROWS, 128), _f32),
            jax.ShapeDtypeStruct((NPAD,), _f32),
            jax.ShapeDtypeStruct((NPAD,), _f32),
            jax.ShapeDtypeStruct((ACCR, D), _f32),
            jax.ShapeDtypeStruct((ACCR, D), _f32),
        ],
        mesh=mesh,
        scratch_types=[
            pltpu.VMEM((CHROWS, 128), _i32),
            pltpu.VMEM((CHROWS, 128), _i32),
            pltpu.VMEM((CHROWS, 128), _i32),
            pltpu.VMEM((CHROWS, 128), _f32),
            pltpu.VMEM((CHROWS, 128), _f32),
            pltpu.VMEM((2, 128, D), _f32),
            pltpu.VMEM_SHARED((NPAD,), _f32),
            pltpu.VMEM_SHARED((ACCR, D), _f32),
            pltpu.SemaphoreType.DMA,
            pltpu.SemaphoreType.DMA,
            pltpu.SemaphoreType.DMA,
        ],
    )


@functools.cache
def _sc_edge_b():
    mesh = plsc.VectorSubcoreMesh(core_axis_name="c", subcore_axis_name="s")
    return pl.kernel(
        _sc_edge_b_body,
        out_type=[
            jax.ShapeDtypeStruct((ACCR, D), _f32),
            jax.ShapeDtypeStruct((ACCR, D), _f32),
        ],
        mesh=mesh,
        scratch_types=[
            pltpu.VMEM((CHROWS, 128), _i32),
            pltpu.VMEM((CHROWS, 128), _i32),
            pltpu.VMEM((CHROWS, 128), _f32),
            pltpu.VMEM((2, 128, D), _f32),
            pltpu.VMEM_SHARED((ACCR, D), _f32),
            pltpu.SemaphoreType.DMA,
            pltpu.SemaphoreType.DMA,
            pltpu.SemaphoreType.DMA,
        ],
    )


def _sc_alpha_body(ee_hbm, ivd_hbm, dst2_hbm, alpha_out, idx_d, eev, ivv, sem):
    cid = lax.axis_index("c")
    sid = lax.axis_index("s")
    wid = sid * 2 + cid
    rbase = wid * CHROWS
    pltpu.sync_copy(dst2_hbm.at[pl.ds(rbase, CHROWS)], idx_d)
    pltpu.sync_copy(ee_hbm.at[pl.ds(rbase, CHROWS)], eev)

    @pl.loop(0, CHROWS, step=8)
    def _(j0):
        cs = []
        for b in range(8):
            cs.append(pltpu.async_copy(ivd_hbm.at[idx_d.at[j0 + b]],
                                       ivv.at[j0 + b], sem))
        for c in cs:
            c.wait()

    @pl.loop(0, CHROWS)
    def _(r):
        for c in range(8):
            sl = pl.ds(c * 16, 16)
            eev[r, sl] = eev[r, sl] * ivv[r, sl]

    pltpu.sync_copy(eev, alpha_out.at[pl.ds(rbase, CHROWS)])


@functools.cache
def _sc_alpha():
    mesh = plsc.VectorSubcoreMesh(core_axis_name="c", subcore_axis_name="s")
    return pl.kernel(
        _sc_alpha_body,
        out_type=jax.ShapeDtypeStruct((NIROWS, 128), _f32),
        mesh=mesh,
        scratch_types=[
            pltpu.VMEM((CHROWS, 128), _i32),
            pltpu.VMEM((CHROWS, 128), _f32),
            pltpu.VMEM((CHROWS, 128), _f32),
            pltpu.SemaphoreType.DMA,
        ],
    )


def kernel(x, We, be, bn_g, bn_b, W_gat, att_src, att_dst, b_gat,
           ln1_g, ln1_b, W1, b1, W2, b2, ln2_g, ln2_b, Wc, bc, edge_index):
    x2 = x.reshape(N, NMOD * IN_DIM)
    w_all = We.reshape(NMOD * IN_DIM, D)

    loop = jnp.arange(N, dtype=edge_index.dtype)
    pad = jnp.zeros((EP - ETOT,), dtype=edge_index.dtype)
    src2 = jnp.concatenate([edge_index[0], loop, pad]).reshape(NIROWS, 128)
    dst2 = jnp.concatenate([edge_index[1], loop, pad]).reshape(NIROWS, 128)

    h_pre, s, ss = _embed(x2, w_all, be)
    h, xw, a_s, a_d = _bn_gat(h_pre, s, ss, bn_g[None, :], bn_b[None, :],
                              W_gat[0], att_src[0].reshape(1, D),
                              att_dst[0].reshape(1, D))

    sc_a = _sc_edge_a()
    sc_b = _sc_edge_b()
    sc_alpha = _sc_alpha()
    alphas = []
    for l in range(L):
        ee, den0, den1, a0, a1 = sc_a(a_s.reshape(N), a_d.reshape(N), src2,
                                      dst2, xw)
        q0, q1 = sc_b(src2, dst2, xw, ee)
        xp0 = jnp.concatenate([a0[:NHALF], q0[:NHALF]], axis=0)
        xp1 = jnp.concatenate([a1[:NHALF], q1[:NHALF]], axis=0)
        args = (h, xp0, xp1, den0[:N, None], den1[:N, None],
                b_gat[l][None, :], ln1_g[l][None, :], ln1_b[l][None, :],
                W1[l], b1[l][None, :], W2[l], b2[l][None, :],
                ln2_g[l][None, :], ln2_b[l][None, :])
        if l < L - 1:
            h, xw, a_s, a_d, ivd = _layer_mid(
                *args, W_gat[l + 1], att_src[l + 1].reshape(1, D),
                att_dst[l + 1].reshape(1, D))
        else:
            logits, ivd = _layer_last(*args, Wc, bc[None, :])
        alpha = sc_alpha(ee, ivd.reshape(N), dst2)
        alphas.append(alpha.reshape(EP)[:ETOT].reshape(ETOT, 1))

    return (logits.reshape(N), alphas[0], alphas[1], alphas[2])
